# Initial kernel scaffold; baseline (speedup 1.0000x reference)
import jax
import jax.numpy as jnp
from jax.experimental import pallas as pl


def kernel(course_tags, user_interests, tag_embedding):
    raise NotImplementedError("write your pallas kernel here")


# speedup vs baseline: 11.8253x; 11.8253x over previous
import functools

import jax
import jax.numpy as jnp
from jax import lax
from jax.experimental import pallas as pl
from jax.experimental.pallas import tpu as pltpu
from jax.experimental.pallas import tpu_sc as plsc

NUM_TAGS = 100000
D = 32
B = 4096
H = 50
L = 16
NC, NS = 2, 16
NW = NC * NS
RPW = B // NW
CH = 2
IPC = CH * H
NCHUNK = RPW // CH


def _sc_kernel(ct_hbm, ui_hbm, table_hbm, out_hbm,
               cidx, uidx, cbuf, ubuf, outv, semc, semu):
    wid = lax.axis_index("s") * NC + lax.axis_index("c")
    crow0 = wid * NCHUNK

    pltpu.sync_copy(ct_hbm.at[pl.ds(crow0, NCHUNK)], cidx)
    pltpu.sync_copy(ui_hbm.at[pl.ds(crow0, NCHUNK)], uidx)

    lane = lax.broadcasted_iota(jnp.int32, (L,), 0)
    last_lane = lane == (L - 1)

    @pl.loop(0, NCHUNK)
    def _chunk(j):
        cc = pltpu.async_copy(table_hbm.at[cidx.at[j]], cbuf, semc)
        uc = pltpu.async_copy(table_hbm.at[uidx.at[j]], ubuf, semu)
        cc.wait()
        uc.wait()
        for i in range(CH):
            c0 = jnp.zeros((L,), jnp.float32)
            c1 = jnp.zeros((L,), jnp.float32)
            u0 = jnp.zeros((L,), jnp.float32)
            u1 = jnp.zeros((L,), jnp.float32)
            for r in range(H):
                row = i * H + r
                c0 = c0 + cbuf[row, pl.ds(0, L)]
                c1 = c1 + cbuf[row, pl.ds(L, L)]
                u0 = u0 + ubuf[row, pl.ds(0, L)]
                u1 = u1 + ubuf[row, pl.ds(L, L)]
            cs = plsc.cumsum(c0 * u0 + c1 * u1) * (1.0 / (H * H))
            row_idx = jnp.full((L,), j * CH + i, jnp.int32)
            plsc.store_scatter(outv, [row_idx], cs, mask=last_lane)

    pltpu.sync_copy(outv, out_hbm.at[pl.ds(wid * RPW, RPW)])


@jax.jit
def kernel(course_tags, user_interests, tag_embedding):
    ct2 = course_tags.reshape(B // CH, IPC).astype(jnp.int32)
    ui2 = user_interests.reshape(B // CH, IPC).astype(jnp.int32)

    run = functools.partial(
        pl.kernel,
        out_type=jax.ShapeDtypeStruct((B,), jnp.float32),
        mesh=plsc.VectorSubcoreMesh(core_axis_name="c", subcore_axis_name="s"),
        compiler_params=pltpu.CompilerParams(
            needs_layout_passes=False, use_tc_tiling_on_sc=False),
        scratch_types=[
            pltpu.VMEM((NCHUNK, IPC), jnp.int32),
            pltpu.VMEM((NCHUNK, IPC), jnp.int32),
            pltpu.VMEM((IPC, D), jnp.float32),
            pltpu.VMEM((IPC, D), jnp.float32),
            pltpu.VMEM((RPW,), jnp.float32),
            pltpu.SemaphoreType.DMA,
            pltpu.SemaphoreType.DMA,
        ],
    )(_sc_kernel)

    sim = run(ct2, ui2, tag_embedding)
    return sim.reshape(B, 1)


# speedup vs baseline: 14.0866x; 1.1912x over previous
import functools

import jax
import jax.numpy as jnp
from jax import lax
from jax.experimental import pallas as pl
from jax.experimental.pallas import tpu as pltpu
from jax.experimental.pallas import tpu_sc as plsc

NUM_TAGS = 100000
D = 32
B = 4096
H = 50
L = 16
NC, NS = 2, 16
NW = NC * NS
RPW = B // NW
CH = 2
IPC = CH * H
NCHUNK = RPW // CH


def _sc_kernel(ct_hbm, ui_hbm, table_hbm, out_hbm,
               cidx, uidx, cbuf0, cbuf1, ubuf0, ubuf1, outv,
               semc0, semc1, semu0, semu1):
    wid = lax.axis_index("s") * NC + lax.axis_index("c")
    crow0 = wid * NCHUNK

    pltpu.sync_copy(ct_hbm.at[pl.ds(crow0, NCHUNK)], cidx)
    pltpu.sync_copy(ui_hbm.at[pl.ds(crow0, NCHUNK)], uidx)

    cbufs, ubufs = (cbuf0, cbuf1), (ubuf0, ubuf1)
    semcs, semus = (semc0, semc1), (semu0, semu1)

    lane = lax.broadcasted_iota(jnp.int32, (L,), 0)
    last_lane = lane == (L - 1)

    def start_set(j, b):
        pltpu.async_copy(table_hbm.at[cidx.at[j]], cbufs[b], semcs[b])
        pltpu.async_copy(table_hbm.at[uidx.at[j]], ubufs[b], semus[b])

    def wait_set(b):
        pltpu.make_async_copy(table_hbm.at[cidx.at[0]], cbufs[b], semcs[b]).wait()
        pltpu.make_async_copy(table_hbm.at[uidx.at[0]], ubufs[b], semus[b]).wait()

    def compute(j, b):
        cb, ub = cbufs[b], ubufs[b]
        for i in range(CH):
            c0 = jnp.zeros((L,), jnp.float32)
            c1 = jnp.zeros((L,), jnp.float32)
            u0 = jnp.zeros((L,), jnp.float32)
            u1 = jnp.zeros((L,), jnp.float32)
            for r in range(H):
                row = i * H + r
                c0 = c0 + cb[row, pl.ds(0, L)]
                c1 = c1 + cb[row, pl.ds(L, L)]
                u0 = u0 + ub[row, pl.ds(0, L)]
                u1 = u1 + ub[row, pl.ds(L, L)]
            cs = plsc.cumsum(c0 * u0 + c1 * u1) * (1.0 / (H * H))
            row_idx = jnp.full((L,), j * CH + i, jnp.int32)
            plsc.store_scatter(outv, [row_idx], cs, mask=last_lane)

    start_set(0, 0)

    @pl.loop(0, NCHUNK, step=2)
    def _chunk(j):
        wait_set(0)
        start_set(j + 1, 1)
        compute(j, 0)
        wait_set(1)

        @pl.when(j + 2 < NCHUNK)
        def _():
            start_set(j + 2, 0)

        compute(j + 1, 1)

    pltpu.sync_copy(outv, out_hbm.at[pl.ds(wid * RPW, RPW)])


@jax.jit
def kernel(course_tags, user_interests, tag_embedding):
    ct2 = course_tags.reshape(B // CH, IPC).astype(jnp.int32)
    ui2 = user_interests.reshape(B // CH, IPC).astype(jnp.int32)

    run = functools.partial(
        pl.kernel,
        out_type=jax.ShapeDtypeStruct((B,), jnp.float32),
        mesh=plsc.VectorSubcoreMesh(core_axis_name="c", subcore_axis_name="s"),
        compiler_params=pltpu.CompilerParams(
            needs_layout_passes=False, use_tc_tiling_on_sc=False),
        scratch_types=[
            pltpu.VMEM((NCHUNK, IPC), jnp.int32),
            pltpu.VMEM((NCHUNK, IPC), jnp.int32),
            pltpu.VMEM((IPC, D), jnp.float32),
            pltpu.VMEM((IPC, D), jnp.float32),
            pltpu.VMEM((IPC, D), jnp.float32),
            pltpu.VMEM((IPC, D), jnp.float32),
            pltpu.VMEM((RPW,), jnp.float32),
            pltpu.SemaphoreType.DMA,
            pltpu.SemaphoreType.DMA,
            pltpu.SemaphoreType.DMA,
            pltpu.SemaphoreType.DMA,
        ],
    )(_sc_kernel)

    sim = run(ct2, ui2, tag_embedding)
    return sim.reshape(B, 1)


# speedup vs baseline: 16.5698x; 1.1763x over previous
import functools

import jax
import jax.numpy as jnp
from jax import lax
from jax.experimental import pallas as pl
from jax.experimental.pallas import tpu as pltpu
from jax.experimental.pallas import tpu_sc as plsc

NUM_TAGS = 100000
D = 32
B = 4096
H = 50
L = 16
NC, NS = 2, 16
NW = NC * NS
RPW = B // NW
NBUF = 4


def _sc_kernel(ct_hbm, ui_hbm, table_hbm, out_hbm,
               cidx, uidx, outv, cbufs, ubufs, semcs, semus):
    wid = lax.axis_index("s") * NC + lax.axis_index("c")
    row0 = wid * RPW

    pltpu.sync_copy(ct_hbm.at[pl.ds(row0, RPW)], cidx)
    pltpu.sync_copy(ui_hbm.at[pl.ds(row0, RPW)], uidx)

    lane = lax.broadcasted_iota(jnp.int32, (L,), 0)
    last_lane = lane == (L - 1)

    def start(j, b):
        pltpu.async_copy(table_hbm.at[cidx.at[j]], cbufs[b], semcs[b])
        pltpu.async_copy(table_hbm.at[uidx.at[j]], ubufs[b], semus[b])

    def wait(b):
        pltpu.make_async_copy(table_hbm.at[cidx.at[0]], cbufs[b], semcs[b]).wait()
        pltpu.make_async_copy(table_hbm.at[uidx.at[0]], ubufs[b], semus[b]).wait()

    def compute(j, b):
        cb, ub = cbufs[b], ubufs[b]
        c0 = jnp.zeros((L,), jnp.float32)
        c1 = jnp.zeros((L,), jnp.float32)
        u0 = jnp.zeros((L,), jnp.float32)
        u1 = jnp.zeros((L,), jnp.float32)
        for r in range(H):
            c0 = c0 + cb[r, pl.ds(0, L)]
            c1 = c1 + cb[r, pl.ds(L, L)]
            u0 = u0 + ub[r, pl.ds(0, L)]
            u1 = u1 + ub[r, pl.ds(L, L)]
        cs = plsc.cumsum(c0 * u0 + c1 * u1) * (1.0 / (H * H))
        row_idx = jnp.full((L,), j, jnp.int32)
        plsc.store_scatter(outv, [row_idx], cs, mask=last_lane)

    for p in range(NBUF - 1):
        start(p, p)

    @pl.loop(0, RPW, step=NBUF)
    def _chunks(j):
        for b in range(NBUF):
            wait(b)

            @pl.when(j + b + NBUF - 1 < RPW)
            def _():
                start(j + b + NBUF - 1, (b + NBUF - 1) % NBUF)

            compute(j + b, b)

    pltpu.sync_copy(outv, out_hbm.at[pl.ds(row0, RPW)])


@jax.jit
def kernel(course_tags, user_interests, tag_embedding):
    ct = course_tags.astype(jnp.int32)
    ui = user_interests.astype(jnp.int32)

    run = functools.partial(
        pl.kernel,
        out_type=jax.ShapeDtypeStruct((B,), jnp.float32),
        mesh=plsc.VectorSubcoreMesh(core_axis_name="c", subcore_axis_name="s"),
        compiler_params=pltpu.CompilerParams(
            needs_layout_passes=False, use_tc_tiling_on_sc=False),
        scratch_types=[
            pltpu.VMEM((RPW, H), jnp.int32),
            pltpu.VMEM((RPW, H), jnp.int32),
            pltpu.VMEM((RPW,), jnp.float32),
            [pltpu.VMEM((H, D), jnp.float32) for _ in range(NBUF)],
            [pltpu.VMEM((H, D), jnp.float32) for _ in range(NBUF)],
            [pltpu.SemaphoreType.DMA for _ in range(NBUF)],
            [pltpu.SemaphoreType.DMA for _ in range(NBUF)],
        ],
    )(_sc_kernel)

    sim = run(ct, ui, tag_embedding)
    return sim.reshape(B, 1)
